# baseline (device time: 12181 ns/iter reference)
import jax
import jax.numpy as jnp
from jax import lax
from jax.experimental import pallas as pl
from jax.experimental.pallas import tpu as pltpu


def kernel(x):
    _, m, n = x.shape
    half = n // 2

    n_chunks = 4
    rows = m // n_chunks

    def body(
        x_hbm,
        out_hbm,
        peer_f32,
        local_f32,
        send_buf,
        recv_buf,
        res_buf,
        pf_sems,
        lf_sems,
        send_sems,
        recv_sems,
        out_sems,
    ):
        mx = lax.axis_index("x")
        my = lax.axis_index("y")
        mz = lax.axis_index("z")
        peer = 1 - mx

        barrier_sem = pltpu.get_barrier_semaphore()
        pl.semaphore_signal(
            barrier_sem,
            inc=1,
            device_id=(peer, my, mz),
            device_id_type=pl.DeviceIdType.MESH,
        )

        peer_fetch = []
        for c in range(n_chunks):
            cp = pltpu.make_async_copy(
                x_hbm.at[0, pl.ds(c * rows, rows), pl.ds(peer * half, half)],
                peer_f32.at[c],
                pf_sems.at[c],
            )
            cp.start()
            peer_fetch.append(cp)
        local_fetch = []
        for c in range(n_chunks):
            cp = pltpu.make_async_copy(
                x_hbm.at[0, pl.ds(c * rows, rows), pl.ds(mx * half, half)],
                local_f32.at[c],
                lf_sems.at[c],
            )
            cp.start()
            local_fetch.append(cp)

        pl.semaphore_wait(barrier_sem, 1)

        rdmas = []
        for c in range(n_chunks):
            peer_fetch[c].wait()
            send_buf[c] = peer_f32[c].astype(jnp.bfloat16)
            rdma = pltpu.make_async_remote_copy(
                src_ref=send_buf.at[c],
                dst_ref=recv_buf.at[c],
                send_sem=send_sems.at[c],
                recv_sem=recv_sems.at[c],
                device_id=(peer, my, mz),
                device_id_type=pl.DeviceIdType.MESH,
            )
            rdma.start()
            rdmas.append(rdma)

        out_copies = []
        for c in range(n_chunks):
            local_fetch[c].wait()
            rdmas[c].wait_recv()
            res_buf[c] = local_f32[c].astype(jnp.bfloat16) + recv_buf[c]
            cp = pltpu.make_async_copy(
                res_buf.at[c],
                out_hbm.at[pl.ds(c * rows, rows), :],
                out_sems.at[c],
            )
            cp.start()
            out_copies.append(cp)

        for c in range(n_chunks):
            out_copies[c].wait()
            rdmas[c].wait_send()

    return pl.pallas_call(
        body,
        out_shape=jax.ShapeDtypeStruct((m, half), jnp.bfloat16),
        in_specs=[pl.BlockSpec(memory_space=pl.ANY)],
        out_specs=pl.BlockSpec(memory_space=pl.ANY),
        scratch_shapes=[
            pltpu.VMEM((n_chunks, rows, half), jnp.float32),
            pltpu.VMEM((n_chunks, rows, half), jnp.float32),
            pltpu.VMEM((n_chunks, rows, half), jnp.bfloat16),
            pltpu.VMEM((n_chunks, rows, half), jnp.bfloat16),
            pltpu.VMEM((n_chunks, rows, half), jnp.bfloat16),
            pltpu.SemaphoreType.DMA((n_chunks,)),
            pltpu.SemaphoreType.DMA((n_chunks,)),
            pltpu.SemaphoreType.DMA((n_chunks,)),
            pltpu.SemaphoreType.DMA((n_chunks,)),
            pltpu.SemaphoreType.DMA((n_chunks,)),
        ],
        compiler_params=pltpu.CompilerParams(collective_id=0),
    )(x)


# device time: 11971 ns/iter; 1.0175x vs baseline; 1.0175x over previous
import jax
import jax.numpy as jnp
from jax import lax
from jax.experimental import pallas as pl
from jax.experimental.pallas import tpu as pltpu


def kernel(x):
    _, m, n = x.shape
    half = n // 2

    n_chunks = 4
    rows = m // n_chunks

    def body(x_ref, out_ref, send_buf, recv_buf, send_sems, recv_sems):
        mx = lax.axis_index("x")
        my = lax.axis_index("y")
        mz = lax.axis_index("z")
        peer = 1 - mx

        barrier_sem = pltpu.get_barrier_semaphore()
        pl.semaphore_signal(
            barrier_sem,
            inc=1,
            device_id=(peer, my, mz),
            device_id_type=pl.DeviceIdType.MESH,
        )

        send_buf[0] = x_ref[0, pl.ds(0, rows), pl.ds(peer * half, half)]
        pl.semaphore_wait(barrier_sem, 1)

        rdmas = []
        for c in range(n_chunks):
            if c > 0:
                send_buf[c] = x_ref[
                    0, pl.ds(c * rows, rows), pl.ds(peer * half, half)
                ]
            rdma = pltpu.make_async_remote_copy(
                src_ref=send_buf.at[c],
                dst_ref=recv_buf.at[c],
                send_sem=send_sems.at[c],
                recv_sem=recv_sems.at[c],
                device_id=(peer, my, mz),
                device_id_type=pl.DeviceIdType.MESH,
            )
            rdma.start()
            rdmas.append(rdma)

        for c in range(n_chunks):
            rdmas[c].wait_recv()
            out_ref[pl.ds(c * rows, rows), :] = (
                x_ref[0, pl.ds(c * rows, rows), pl.ds(mx * half, half)]
                + recv_buf[c]
            )

        for c in range(n_chunks):
            rdmas[c].wait_send()

    return pl.pallas_call(
        body,
        out_shape=jax.ShapeDtypeStruct((m, half), jnp.bfloat16),
        in_specs=[pl.BlockSpec(memory_space=pltpu.VMEM)],
        out_specs=pl.BlockSpec(memory_space=pltpu.VMEM),
        scratch_shapes=[
            pltpu.VMEM((n_chunks, rows, half), jnp.bfloat16),
            pltpu.VMEM((n_chunks, rows, half), jnp.bfloat16),
            pltpu.SemaphoreType.DMA((n_chunks,)),
            pltpu.SemaphoreType.DMA((n_chunks,)),
        ],
        compiler_params=pltpu.CompilerParams(collective_id=0),
    )(x.astype(jnp.bfloat16))


# device time: 11963 ns/iter; 1.0182x vs baseline; 1.0007x over previous
import jax
import jax.numpy as jnp
from jax import lax
from jax.experimental import pallas as pl
from jax.experimental.pallas import tpu as pltpu


def kernel(x):
    _, m, n = x.shape
    half = n // 2

    n_chunks = 4
    rows = m // n_chunks

    def body(x_ref, out_ref, send_buf, recv_buf, send_sems, recv_sems):
        mx = lax.axis_index("x")
        my = lax.axis_index("y")
        mz = lax.axis_index("z")
        peer = 1 - mx

        barrier_sem = pltpu.get_barrier_semaphore()
        pl.semaphore_signal(
            barrier_sem,
            inc=1,
            device_id=(peer, my, mz),
            device_id_type=pl.DeviceIdType.MESH,
        )

        pl.semaphore_wait(barrier_sem, 1)

        rdmas = []
        for c in range(n_chunks):
            row = pl.ds(c * rows, rows)

            @pl.when(mx == 0)
            def _(c=c, row=row):
                send_buf[c] = x_ref[row, pl.ds(half, half)]

            @pl.when(mx == 1)
            def _(c=c, row=row):
                send_buf[c] = x_ref[row, pl.ds(0, half)]

            rdma = pltpu.make_async_remote_copy(
                src_ref=send_buf.at[c],
                dst_ref=recv_buf.at[c],
                send_sem=send_sems.at[c],
                recv_sem=recv_sems.at[c],
                device_id=(peer, my, mz),
                device_id_type=pl.DeviceIdType.MESH,
            )
            rdma.start()
            rdmas.append(rdma)

        for c in range(n_chunks):
            rdmas[c].wait_recv()
            row = pl.ds(c * rows, rows)

            @pl.when(mx == 0)
            def _(c=c, row=row):
                out_ref[row, :] = x_ref[row, pl.ds(0, half)] + recv_buf[c]

            @pl.when(mx == 1)
            def _(c=c, row=row):
                out_ref[row, :] = (
                    x_ref[row, pl.ds(half, half)] + recv_buf[c]
                )

        for c in range(n_chunks):
            rdmas[c].wait_send()

    return pl.pallas_call(
        body,
        out_shape=jax.ShapeDtypeStruct((m, half), jnp.bfloat16),
        in_specs=[pl.BlockSpec(memory_space=pltpu.VMEM)],
        out_specs=pl.BlockSpec(memory_space=pltpu.VMEM),
        scratch_shapes=[
            pltpu.VMEM((n_chunks, rows, half), jnp.bfloat16),
            pltpu.VMEM((n_chunks, rows, half), jnp.bfloat16),
            pltpu.SemaphoreType.DMA((n_chunks,)),
            pltpu.SemaphoreType.DMA((n_chunks,)),
        ],
        compiler_params=pltpu.CompilerParams(collective_id=0),
    )(x[0].astype(jnp.bfloat16))


# device time: 11326 ns/iter; 1.0755x vs baseline; 1.0562x over previous
import jax
import jax.numpy as jnp
from jax import lax
from jax.experimental import pallas as pl
from jax.experimental.pallas import tpu as pltpu


def kernel(x):
    _, m, n = x.shape
    half = n // 2

    n_chunks = 4
    rows = m // n_chunks

    def body(
        x_hbm,
        out_ref,
        peer_f32,
        local_f32,
        send_buf,
        recv_buf,
        pf_sems,
        lf_sems,
        send_sems,
        recv_sems,
    ):
        mx = lax.axis_index("x")
        my = lax.axis_index("y")
        mz = lax.axis_index("z")
        peer = 1 - mx

        barrier_sem = pltpu.get_barrier_semaphore()
        pl.semaphore_signal(
            barrier_sem,
            inc=1,
            device_id=(peer, my, mz),
            device_id_type=pl.DeviceIdType.MESH,
        )

        peer_fetch = []
        for c in range(n_chunks):
            cp = pltpu.make_async_copy(
                x_hbm.at[pl.ds(c * rows, rows), pl.ds(peer * half, half)],
                peer_f32.at[c],
                pf_sems.at[c],
            )
            cp.start()
            peer_fetch.append(cp)
        local_fetch = []
        for c in range(n_chunks):
            cp = pltpu.make_async_copy(
                x_hbm.at[pl.ds(c * rows, rows), pl.ds(mx * half, half)],
                local_f32.at[c],
                lf_sems.at[c],
            )
            cp.start()
            local_fetch.append(cp)

        pl.semaphore_wait(barrier_sem, 1)

        rdmas = []
        for c in range(n_chunks):
            peer_fetch[c].wait()
            send_buf[c] = peer_f32[c].astype(jnp.bfloat16)
            rdma = pltpu.make_async_remote_copy(
                src_ref=send_buf.at[c],
                dst_ref=recv_buf.at[c],
                send_sem=send_sems.at[c],
                recv_sem=recv_sems.at[c],
                device_id=(peer, my, mz),
                device_id_type=pl.DeviceIdType.MESH,
            )
            rdma.start()
            rdmas.append(rdma)

        for c in range(n_chunks):
            local_fetch[c].wait()
            rdmas[c].wait_recv()
            out_ref[pl.ds(c * rows, rows), :] = (
                local_f32[c].astype(jnp.bfloat16) + recv_buf[c]
            )

        for c in range(n_chunks):
            rdmas[c].wait_send()

    x2 = pltpu.with_memory_space_constraint(x[0], pltpu.MemorySpace.HBM)
    return pl.pallas_call(
        body,
        out_shape=jax.ShapeDtypeStruct((m, half), jnp.bfloat16),
        in_specs=[pl.BlockSpec(memory_space=pl.ANY)],
        out_specs=pl.BlockSpec(memory_space=pltpu.VMEM),
        scratch_shapes=[
            pltpu.VMEM((n_chunks, rows, half), jnp.float32),
            pltpu.VMEM((n_chunks, rows, half), jnp.float32),
            pltpu.VMEM((n_chunks, rows, half), jnp.bfloat16),
            pltpu.VMEM((n_chunks, rows, half), jnp.bfloat16),
            pltpu.SemaphoreType.DMA((n_chunks,)),
            pltpu.SemaphoreType.DMA((n_chunks,)),
            pltpu.SemaphoreType.DMA((n_chunks,)),
            pltpu.SemaphoreType.DMA((n_chunks,)),
        ],
        compiler_params=pltpu.CompilerParams(collective_id=0),
    )(x2)
